# Initial kernel scaffold; baseline (speedup 1.0000x reference)
import jax
import jax.numpy as jnp
from jax.experimental import pallas as pl


def kernel(x, edge_index, params):
    raise NotImplementedError("write your pallas kernel here")


# speedup vs baseline: 112.3271x; 112.3271x over previous
import functools

import jax
import jax.numpy as jnp
from jax import lax
from jax.experimental import pallas as pl
from jax.experimental.pallas import tpu as pltpu
from jax.experimental.pallas import tpu_sc as plsc

N = 512
E = 131072
NC = 2
NS = 16
NW = NC * NS
EPT = E // NW
CHUNK = 128
NCHUNK = EPT // CHUNK
ZSL = N * N // NS


def _sc_body(src_hbm, dst_hbm, zeros_hbm, out_hbm, src_v, dst_v, idx_v,
             ones_v, a_sh):
    c = lax.axis_index("c")
    s = lax.axis_index("s")
    wid = s * NC + c

    pltpu.sync_copy(zeros_hbm.at[pl.ds(s * ZSL, ZSL)],
                    a_sh.at[pl.ds(s * ZSL, ZSL)])

    base = wid * EPT
    pltpu.sync_copy(src_hbm.at[pl.ds(base, EPT)], src_v)
    pltpu.sync_copy(dst_hbm.at[pl.ds(base, EPT)], dst_v)

    for j in range(CHUNK // 16):
        ones_v[pl.ds(j * 16, 16)] = jnp.ones((16,), jnp.float32)

    def idx_body(t, carry):
        sv = src_v[pl.ds(t * 16, 16)]
        dv = dst_v[pl.ds(t * 16, 16)]
        row = t // (CHUNK // 16)
        col = (t % (CHUNK // 16)) * 16
        idx_v[row, pl.ds(col, 16)] = dv * N + sv
        return carry

    lax.fori_loop(0, EPT // 16, idx_body, 0)
    plsc.subcore_barrier()

    def scat_body(k, carry):
        pltpu.sync_copy(ones_v, a_sh.at[idx_v.at[k]], add=True)
        return carry

    lax.fori_loop(0, NCHUNK, scat_body, 0)
    plsc.subcore_barrier()

    off = c * (N * N) + s * ZSL
    pltpu.sync_copy(a_sh.at[pl.ds(s * ZSL, ZSL)], out_hbm.at[pl.ds(off, ZSL)])


@functools.cache
def _sc_build_fn():
    return pl.kernel(
        _sc_body,
        out_type=jax.ShapeDtypeStruct((NC * N * N,), jnp.float32),
        mesh=plsc.VectorSubcoreMesh(core_axis_name="c", subcore_axis_name="s",
                                    num_cores=NC, num_subcores=NS),
        scratch_types=[
            pltpu.VMEM((EPT,), jnp.int32),
            pltpu.VMEM((EPT,), jnp.int32),
            pltpu.VMEM((NCHUNK, CHUNK), jnp.int32),
            pltpu.VMEM((CHUNK,), jnp.float32),
            pltpu.VMEM_SHARED((N * N,), jnp.float32),
        ],
        name="edge_count_scatter",
    )


def _tc_body(a_ref, x_ref, w1, b1, w2, b2,
             wi1, wc1, wo1, cbi1, cbc1, cbo1, bi1, bc1, bo1, wco1,
             wi2, wc2, wo2, cbi2, cbc2, cbo2, bi2, bc2, bo2, wco2,
             linw, linb, out_ref):
    f32 = jnp.float32
    A = a_ref[0] + a_ref[1]
    deg = jnp.sum(A, axis=1, keepdims=True) + 2.0
    dinv = lax.rsqrt(deg)
    sc2 = 2.0 * dinv * dinv

    def conv(h, w, b):
        u = jnp.dot(h, w[...], preferred_element_type=f32)
        t = jnp.dot(A, dinv * u, preferred_element_type=f32)
        return dinv * t + sc2 * u + b[...]

    h = jax.nn.relu(conv(x_ref[...], w1, b1))
    h = jax.nn.relu(conv(h, w2, b2))

    def cell(h, wi, wc, wo, cbi, cbc, cbo, bi, bc, bo, wco):
        gi = jax.nn.sigmoid(
            jnp.dot(h, wi[...], preferred_element_type=f32) + cbi[...] + bi[...])
        gt = jnp.tanh(
            jnp.dot(h, wc[...], preferred_element_type=f32) + cbc[...] + bc[...])
        cst = gi * gt
        go = jax.nn.sigmoid(
            jnp.dot(h, wo[...], preferred_element_type=f32)
            + cbo[...] + wco[...] * cst + bo[...])
        return go * jnp.tanh(cst)

    h = cell(h, wi1, wc1, wo1, cbi1, cbc1, cbo1, bi1, bc1, bo1, wco1)
    h = cell(h, wi2, wc2, wo2, cbi2, cbc2, cbo2, bi2, bc2, bo2, wco2)
    out_ref[...] = (jnp.dot(jax.nn.relu(h), linw[...], preferred_element_type=f32)
                    + linb[...])


def kernel(x, edge_index, params):
    p = params
    src = edge_index[0].astype(jnp.int32)
    dst = edge_index[1].astype(jnp.int32)
    zeros_flat = jnp.zeros((N * N,), jnp.float32)
    a_parts = _sc_build_fn()(src, dst, zeros_flat).reshape(NC, N, N)

    row = lambda v: v.reshape(1, N)
    lin_w = jnp.concatenate(
        [p['lin_W'], jnp.zeros((N, 128 - p['lin_W'].shape[1]), jnp.float32)], axis=1)
    lin_b = jnp.concatenate(
        [p['lin_b'], jnp.zeros((128 - p['lin_b'].shape[0],), jnp.float32)]).reshape(1, 128)

    args = [a_parts, x,
            p['gcn1_W'], row(p['gcn1_b']), p['gcn2_W'], row(p['gcn2_b'])]
    for cell_name in ('lstm1', 'lstm2'):
        args += [p[cell_name + '_W_i'], p[cell_name + '_W_c'], p[cell_name + '_W_o'],
                 row(p[cell_name + '_cb_i']), row(p[cell_name + '_cb_c']),
                 row(p[cell_name + '_cb_o']),
                 p[cell_name + '_b_i'], p[cell_name + '_b_c'], p[cell_name + '_b_o'],
                 p[cell_name + '_wc_o']]
    args += [lin_w, lin_b]

    out = pl.pallas_call(
        _tc_body,
        out_shape=jax.ShapeDtypeStruct((N, 128), jnp.float32),
    )(*args)
    return out[:, :p['lin_W'].shape[1]]
